# baseline (speedup 1.0000x reference)
import functools

import jax
import jax.numpy as jnp
from jax import lax
from jax.experimental import pallas as pl
from jax.experimental.pallas import tpu as pltpu
from jax.experimental.pallas import tpu_sc as plsc

_F32 = jnp.float32
_BF16 = jnp.bfloat16


def _gelu(x):
    return x * 0.5 * (1.0 + lax.erf(x * (1.0 / jnp.sqrt(2.0).astype(_F32))))


def _ln(x, g, b, eps=1e-5):
    m = jnp.mean(x, axis=-1, keepdims=True)
    v = jnp.mean((x - m) ** 2, axis=-1, keepdims=True)
    return (x - m) / jnp.sqrt(v + eps) * g + b


def _bdot(a, w):
    return jnp.dot(a.astype(_BF16), w, preferred_element_type=_F32)


def _prep_body(hv_ref, w_ref, b_ref, a_ref, n_ref):
    h = hv_ref.shape[1]
    x = jnp.dot(hv_ref[...], w_ref[...], preferred_element_type=_F32)
    a_ref[...] = x[:, :h] + b_ref[...]
    n_ref[...] = x[:, h:].astype(_BF16)


def _prep(h_v, w_cat, b1):
    l, h = h_v.shape
    return pl.pallas_call(
        _prep_body,
        out_shape=(
            jax.ShapeDtypeStruct((l, h), _F32),
            jax.ShapeDtypeStruct((l, h), _BF16),
        ),
    )(h_v, w_cat, b1)


def _sc_gather(table, idx_flat):
    l, w = table.shape
    dt = table.dtype
    e = idx_flat.shape[0]
    nc, ns = 2, 16
    nw = nc * ns
    rpw = e // nw
    ch = 400
    nch = rpw // ch
    mesh = plsc.VectorSubcoreMesh(core_axis_name="c", subcore_axis_name="s")

    @functools.partial(
        pl.kernel,
        mesh=mesh,
        compiler_params=pltpu.CompilerParams(use_tc_tiling_on_sc=False),
        out_type=jax.ShapeDtypeStruct((e, w), dt),
        scratch_types=[
            pltpu.VMEM((ch,), jnp.int32),
            pltpu.VMEM((ch,), jnp.int32),
            pltpu.VMEM((ch, w), dt),
            pltpu.VMEM((ch, w), dt),
            pltpu.SemaphoreType.DMA,
            pltpu.SemaphoreType.DMA,
        ],
    )
    def k(table_hbm, idx_hbm, out_hbm, idx_v0, idx_v1, rows_v0, rows_v1,
          sem0, sem1):
        wid = lax.axis_index("s") * nc + lax.axis_index("c")
        base = wid * rpw
        idx_v = (idx_v0, idx_v1)
        rows_v = (rows_v0, rows_v1)
        sems = (sem0, sem1)

        def start(s):
            b = s % 2
            pltpu.sync_copy(idx_hbm.at[pl.ds(base + s * ch, ch)], idx_v[b])
            return pltpu.async_copy(table_hbm.at[idx_v[b]], rows_v[b], sems[b])

        pending = start(0)
        for s in range(nch):
            nxt = start(s + 1) if s + 1 < nch else None
            pending.wait()
            pltpu.sync_copy(rows_v[s % 2], out_hbm.at[pl.ds(base + s * ch, ch)])
            pending = nxt

    return k(table, idx_flat)


def _gather_bf16(table_bf, idx_flat):
    l, h = table_bf.shape
    t_i32 = lax.bitcast_convert_type(table_bf.reshape(l, h // 2, 2), jnp.int32)
    g_i32 = _sc_gather(t_i32, idx_flat)
    e = idx_flat.shape[0]
    return lax.bitcast_convert_type(g_i32, _BF16).reshape(e, h)


def _tc1_body(hE, G, A, hV, mam, mv,
              w1e, w2, b2, w3, b3, g1, o1,
              win, bin_, wout, bout, g2, o2,
              w11cat, b11,
              hvo, a2o, n2o):
    bn, h = hV.shape
    bnk = hE.shape[0]
    k = bnk // bn
    x = _bdot(hE[...], w1e[...]) + G[...].astype(_F32)
    x = x.reshape(bn, k, h) + A[...].reshape(bn, 1, h)
    m = _gelu(x).reshape(bnk, h)
    m = _gelu(_bdot(m, w2[...]) + b2[...])
    m = _bdot(m, w3[...]) + b3[...]
    m = m * mam[...]
    dh = jnp.sum(m.reshape(bn, k, h), axis=1) / 30.0
    v = _ln(hV[...] + dh, g1[...], o1[...])
    f = jnp.dot(_gelu(jnp.dot(v, win[...], preferred_element_type=_F32) + bin_[...]),
                wout[...], preferred_element_type=_F32) + bout[...]
    v = _ln(v + f, g2[...], o2[...])
    v = v * mv[...]
    hvo[...] = v
    y = jnp.dot(v, w11cat[...], preferred_element_type=_F32)
    a2o[...] = y[:, :h] + b11[...]
    n2o[...] = y[:, h:].astype(_BF16)


def _tc2_body(hE, G, A, w11e, w12, b12, w13, b13, g3, o3, hEo):
    bn, h = A.shape
    bnk = hE.shape[0]
    k = bnk // bn
    x = _bdot(hE[...], w11e[...]) + G[...].astype(_F32)
    x = x.reshape(bn, k, h) + A[...].reshape(bn, 1, h)
    m = _gelu(x).reshape(bnk, h)
    m = _gelu(_bdot(m, w12[...]) + b12[...])
    m = _bdot(m, w13[...]) + b13[...]
    hEo[...] = _ln(hE[...] + m, g3[...], o3[...])


def kernel(h_V, h_E, E_idx, mask_V, mask_attend, params):
    p = params
    l, h = h_V.shape
    k = E_idx.shape[1]
    bn = 400
    grid = l // bn

    hE2 = h_E.reshape(l * k, h)
    idx = E_idx.reshape(l * k)
    mam = mask_attend.reshape(l * k, 1)
    mv = mask_V.reshape(l, 1)

    def r1(a):
        return a.reshape(1, -1)

    def b16(a):
        return a.astype(_BF16)

    w1s, w1e, w1n = p['W1'][:h], p['W1'][h:2 * h], p['W1'][2 * h:]
    w11s, w11e, w11n = p['W11'][:h], p['W11'][h:2 * h], p['W11'][2 * h:]

    a1, n1 = _prep(h_V, jnp.concatenate([w1s, w1n], axis=1), r1(p['b1']))
    g1 = _gather_bf16(n1, idx)

    node_spec = pl.BlockSpec((bn, h), lambda i: (i, 0))
    edge_spec = pl.BlockSpec((bn * k, h), lambda i: (i, 0))
    full = lambda arr: pl.BlockSpec(arr.shape, lambda i: tuple(0 for _ in arr.shape))
    w11cat = jnp.concatenate([w11s, w11n], axis=1)
    ins1 = (hE2, g1, a1, h_V, mam, mv,
            b16(w1e), b16(p['W2']), r1(p['b2']), b16(p['W3']), r1(p['b3']),
            r1(p['g1']), r1(p['o1']),
            p['W_in'], r1(p['b_in']), p['W_out'], r1(p['b_out']), r1(p['g2']), r1(p['o2']),
            w11cat, r1(p['b11']))
    specs1 = [edge_spec, edge_spec, node_spec, node_spec,
              pl.BlockSpec((bn * k, 1), lambda i: (i, 0)),
              pl.BlockSpec((bn, 1), lambda i: (i, 0))] + [full(a) for a in ins1[6:]]
    hvo, a2, n2 = pl.pallas_call(
        _tc1_body,
        grid=(grid,),
        in_specs=specs1,
        out_specs=(node_spec, node_spec, node_spec),
        out_shape=(jax.ShapeDtypeStruct((l, h), _F32),
                   jax.ShapeDtypeStruct((l, h), _F32),
                   jax.ShapeDtypeStruct((l, h), _BF16)),
    )(*ins1)

    g2_ = _gather_bf16(n2, idx)

    ins2 = (hE2, g2_, a2, b16(w11e), b16(p['W12']), r1(p['b12']), b16(p['W13']),
            r1(p['b13']), r1(p['g3']), r1(p['o3']))
    specs2 = [edge_spec, edge_spec, node_spec] + [full(a) for a in ins2[3:]]
    heo = pl.pallas_call(
        _tc2_body,
        grid=(grid,),
        in_specs=specs2,
        out_specs=edge_spec,
        out_shape=jax.ShapeDtypeStruct((l * k, h), _F32),
    )(*ins2)

    return hvo, heo.reshape(l, k, h)


# speedup vs baseline: 3.4076x; 3.4076x over previous
import functools

import jax
import jax.numpy as jnp
from jax import lax
from jax.experimental import pallas as pl
from jax.experimental.pallas import tpu as pltpu
from jax.experimental.pallas import tpu_sc as plsc

_F32 = jnp.float32
_BF16 = jnp.bfloat16


def _gelu(x):
    return x * 0.5 * (1.0 + lax.erf(x * (1.0 / jnp.sqrt(2.0).astype(_F32))))


def _ln(x, g, b, eps=1e-5):
    m = jnp.mean(x, axis=-1, keepdims=True)
    v = jnp.mean((x - m) ** 2, axis=-1, keepdims=True)
    return (x - m) / jnp.sqrt(v + eps) * g + b


def _ln_mxu(x, g, b, eps=1e-5):
    h = x.shape[-1]
    ones = jnp.full((h, h), 1.0 / h, dtype=_F32)
    m = jnp.dot(x, ones, preferred_element_type=_F32)
    d = x - m
    v = jnp.dot(d * d, ones, preferred_element_type=_F32)
    return d * lax.rsqrt(v + eps) * g + b


def _bdot(a, w):
    return jnp.dot(a.astype(_BF16), w, preferred_element_type=_F32)


def _prep_body(hv_ref, w_ref, b_ref, a_ref, n_ref):
    h = hv_ref.shape[1]
    x = jnp.dot(hv_ref[...], w_ref[...], preferred_element_type=_F32)
    a_ref[...] = x[:, :h] + b_ref[...]
    n_ref[...] = x[:, h:]


def _prep(h_v, w_cat, b1):
    l, h = h_v.shape
    return pl.pallas_call(
        _prep_body,
        out_shape=(
            jax.ShapeDtypeStruct((l, h), _F32),
            jax.ShapeDtypeStruct((l, h), _F32),
        ),
    )(h_v, w_cat, b1)


def _sc_gather(table, idx_flat):
    l, w = table.shape
    dt = table.dtype
    e = idx_flat.shape[0]
    nc, ns = 2, 16
    nw = nc * ns
    rpw = e // nw
    ch = 400
    nch = rpw // ch
    mesh = plsc.VectorSubcoreMesh(core_axis_name="c", subcore_axis_name="s")

    @functools.partial(
        pl.kernel,
        mesh=mesh,
        out_type=jax.ShapeDtypeStruct((e, w), dt),
        scratch_types=[
            pltpu.VMEM((ch,), jnp.int32),
            pltpu.VMEM((ch,), jnp.int32),
            pltpu.VMEM((ch, w), dt),
            pltpu.VMEM((ch, w), dt),
            pltpu.SemaphoreType.DMA,
            pltpu.SemaphoreType.DMA,
        ],
    )
    def k(table_hbm, idx_hbm, out_hbm, idx_v0, idx_v1, rows_v0, rows_v1,
          sem0, sem1):
        wid = lax.axis_index("s") * nc + lax.axis_index("c")
        base = wid * rpw
        idx_v = (idx_v0, idx_v1)
        rows_v = (rows_v0, rows_v1)
        sems = (sem0, sem1)

        def start(s):
            b = s % 2
            pltpu.sync_copy(idx_hbm.at[pl.ds(base + s * ch, ch)], idx_v[b])
            return pltpu.async_copy(table_hbm.at[idx_v[b]], rows_v[b], sems[b])

        pending = start(0)
        for s in range(nch):
            nxt = start(s + 1) if s + 1 < nch else None
            pending.wait()
            pltpu.sync_copy(rows_v[s % 2], out_hbm.at[pl.ds(base + s * ch, ch)])
            pending = nxt

    return k(table, idx_flat)


def _gather_bf16(table_bf, idx_flat):
    l, h = table_bf.shape
    t_i32 = lax.bitcast_convert_type(table_bf.reshape(l, h // 2, 2), jnp.int32)
    g_i32 = _sc_gather(t_i32, idx_flat)
    e = idx_flat.shape[0]
    return lax.bitcast_convert_type(g_i32, _BF16).reshape(e, h)


def _tc1_body(hE, G, A, hV, mam, ma2, mv,
              w1e, w2, b2, w3, b3, g1, o1,
              win, bin_, wout, bout, g2, o2,
              w11cat, b11,
              hvo, a2o, n2o):
    bn, h = hV.shape
    bnk = hE.shape[0]
    k = bnk // bn
    x = jnp.dot(hE[...], w1e[...], preferred_element_type=_F32) + G[...]
    x = x.reshape(bn, k, h) + A[...].reshape(bn, 1, h)
    q = _gelu(x).reshape(bnk, h)
    q = _gelu(jnp.dot(q, w2[...], preferred_element_type=_F32) + b2[...])
    q = q * mam[...]
    sq = jnp.sum(q.reshape(bn, k, h), axis=1)
    cnt = jnp.sum(ma2[...], axis=1, keepdims=True)
    dh = (jnp.dot(sq, w3[...], preferred_element_type=_F32) + cnt * b3[...]) / 30.0
    v = _ln(hV[...] + dh, g1[...], o1[...])
    f = jnp.dot(_gelu(jnp.dot(v, win[...], preferred_element_type=_F32) + bin_[...]),
                wout[...], preferred_element_type=_F32) + bout[...]
    v = _ln(v + f, g2[...], o2[...])
    v = v * mv[...]
    hvo[...] = v
    y = jnp.dot(v, w11cat[...], preferred_element_type=_F32)
    a2o[...] = y[:, :h] + b11[...]
    n2o[...] = y[:, h:]


def _tc2_body(hE, G, A, w11e, w12, b12, w13, b13, g3, o3, hEo):
    bn, h = A.shape
    bnk = hE.shape[0]
    k = bnk // bn
    x = jnp.dot(hE[...], w11e[...], preferred_element_type=_F32) + G[...]
    x = x.reshape(bn, k, h) + A[...].reshape(bn, 1, h)
    m = _gelu(x).reshape(bnk, h)
    m = _gelu(jnp.dot(m, w12[...], preferred_element_type=_F32) + b12[...])
    m = jnp.dot(m, w13[...], preferred_element_type=_F32) + b13[...]
    hEo[...] = _ln_mxu(hE[...] + m, g3[...], o3[...])


def kernel(h_V, h_E, E_idx, mask_V, mask_attend, params):
    p = params
    l, h = h_V.shape
    k = E_idx.shape[1]
    bn = 400
    grid = l // bn

    hE2 = h_E.reshape(l * k, h)
    idx = E_idx.reshape(l * k)
    mam = mask_attend.reshape(l * k, 1)
    mv = mask_V.reshape(l, 1)

    def r1(a):
        return a.reshape(1, -1)

    def b16(a):
        return a.astype(_BF16)

    w1s, w1e, w1n = p['W1'][:h], p['W1'][h:2 * h], p['W1'][2 * h:]
    w11s, w11e, w11n = p['W11'][:h], p['W11'][h:2 * h], p['W11'][2 * h:]

    a1, n1 = _prep(h_V, jnp.concatenate([w1s, w1n], axis=1), r1(p['b1']))
    g1 = _sc_gather(n1, idx)

    node_spec = pl.BlockSpec((bn, h), lambda i: (i, 0))
    edge_spec = pl.BlockSpec((bn * k, h), lambda i: (i, 0))
    full = lambda arr: pl.BlockSpec(arr.shape, lambda i: tuple(0 for _ in arr.shape))
    w11cat = jnp.concatenate([w11s, w11n], axis=1)
    ins1 = (hE2, g1, a1, h_V, mam, mask_attend, mv,
            w1e, p['W2'], r1(p['b2']), p['W3'], r1(p['b3']),
            r1(p['g1']), r1(p['o1']),
            p['W_in'], r1(p['b_in']), p['W_out'], r1(p['b_out']), r1(p['g2']), r1(p['o2']),
            w11cat, r1(p['b11']))
    specs1 = [edge_spec, edge_spec, node_spec, node_spec,
              pl.BlockSpec((bn * k, 1), lambda i: (i, 0)),
              pl.BlockSpec((bn, k), lambda i: (i, 0)),
              pl.BlockSpec((bn, 1), lambda i: (i, 0))] + [full(a) for a in ins1[7:]]
    hvo, a2, n2 = pl.pallas_call(
        _tc1_body,
        grid=(grid,),
        in_specs=specs1,
        out_specs=(node_spec, node_spec, node_spec),
        out_shape=(jax.ShapeDtypeStruct((l, h), _F32),
                   jax.ShapeDtypeStruct((l, h), _F32),
                   jax.ShapeDtypeStruct((l, h), _F32)),
    )(*ins1)

    g2_ = _sc_gather(n2, idx)

    ins2 = (hE2, g2_, a2, w11e, p['W12'], r1(p['b12']), p['W13'],
            r1(p['b13']), r1(p['g3']), r1(p['o3']))
    specs2 = [edge_spec, edge_spec, node_spec] + [full(a) for a in ins2[3:]]
    heo = pl.pallas_call(
        _tc2_body,
        grid=(grid,),
        in_specs=specs2,
        out_specs=edge_spec,
        out_shape=jax.ShapeDtypeStruct((l * k, h), _F32),
    )(*ins2)

    return hvo, heo.reshape(l, k, h)
